# baseline (speedup 1.0000x reference)
import functools

import jax
import jax.numpy as jnp
from jax import lax
from jax.experimental import pallas as pl
from jax.experimental.pallas import tpu as pltpu
from jax.experimental.pallas import tpu_sc as plsc

_N = 50000
_E = 800000
_G = 64
_EPAD = 819200
_NROW = _EPAD // 128
_ACC_R = 50048
_B = 2000
_GRID = _N // _B

_mesh = plsc.VectorSubcoreMesh(
    core_axis_name="c", subcore_axis_name="s", num_cores=2, num_subcores=16)


def _make_seg(feature_split):
    ch = 10
    if feature_split:
        rows_per_tile = _NROW // 16
        npass = 2
    else:
        rows_per_tile = _NROW // 32
        npass = 1
    nch = rows_per_tile // ch
    zrows = _ACC_R // 16
    orows = 3128
    obase_last = _N - orows

    def body(zeros_h, src_h, dst_h, tab_h, out_h, idx_s, idx_d, rows, acc,
             gsem, ssem):
        c = lax.axis_index("c")
        s = lax.axis_index("s")
        for p in range(npass):
            if feature_split:
                part = c * npass + p
                src_base = part * _NROW + s * rows_per_tile
                dst_base = s * rows_per_tile
            else:
                part = c
                src_base = (c * 16 + s) * rows_per_tile
                dst_base = src_base
            if p > 0:
                plsc.subcore_barrier()
            pltpu.sync_copy(zeros_h.at[pl.ds(s * zrows, zrows)],
                            acc.at[pl.ds(s * zrows, zrows)])
            plsc.subcore_barrier()

            pltpu.sync_copy(src_h.at[pl.ds(src_base, ch)], idx_s.at[0])
            pltpu.sync_copy(dst_h.at[pl.ds(dst_base, ch)], idx_d.at[0])
            for j in range(ch):
                pltpu.async_copy(tab_h.at[idx_s.at[0, j]], rows.at[0, j], gsem)

            @pl.loop(0, nch)
            def _chunk(k, src_base=src_base, dst_base=dst_base):
                pb = lax.rem(k, 2)
                pn = lax.rem(k + 1, 2)
                for j in range(ch):
                    pltpu.make_async_copy(tab_h.at[idx_s.at[pb, j]],
                                          rows.at[pb, j], gsem).wait()
                sdescs = [pltpu.async_copy(rows.at[pb, j],
                                           acc.at[idx_d.at[pb, j]], ssem,
                                           add=True) for j in range(ch)]

                @pl.when(k < nch - 1)
                def _():
                    pltpu.sync_copy(
                        src_h.at[pl.ds(src_base + (k + 1) * ch, ch)],
                        idx_s.at[pn])
                    pltpu.sync_copy(
                        dst_h.at[pl.ds(dst_base + (k + 1) * ch, ch)],
                        idx_d.at[pn])
                    for j in range(ch):
                        pltpu.async_copy(tab_h.at[idx_s.at[pn, j]],
                                         rows.at[pn, j], gsem)

                for d in sdescs:
                    d.wait()

            plsc.subcore_barrier()
            ob = jnp.minimum(s * orows, obase_last)
            pltpu.sync_copy(acc.at[pl.ds(ob, orows)],
                            out_h.at[pl.ds(part * _N + ob, orows)])

    return pl.kernel(
        body,
        out_type=jax.ShapeDtypeStruct(((4 if feature_split else 2) * _N, 16),
                                      jnp.float32),
        mesh=_mesh,
        scratch_types=[
            pltpu.VMEM((2, ch, 128), jnp.int32),
            pltpu.VMEM((2, ch, 128), jnp.int32),
            pltpu.VMEM((2, ch, 128, 16), jnp.float32),
            pltpu.VMEM_SHARED((_ACC_R, 16), jnp.float32),
            pltpu.SemaphoreType.DMA,
            pltpu.SemaphoreType.DMA,
        ],
        compiler_params=pltpu.CompilerParams(use_tc_tiling_on_sc=False),
    )


_seg64 = _make_seg(feature_split=True)
_seg16 = _make_seg(feature_split=False)


def _dot(a, b):
    return jnp.dot(a, b, preferred_element_type=jnp.float32)


def _layer1_body(xin_ref, mp_ref, ws_ref, wn_ref, b_ref, x1_ref, rdeg_ref):
    m = mp_ref[0] + mp_ref[1]
    deg = jnp.maximum(m[:, 2:3], 1.0)
    agg = m[:, 0:2] / deg
    h = _dot(xin_ref[...], ws_ref[...]) + _dot(agg, wn_ref[...]) + b_ref[...]
    h = jnp.maximum(h, 0.0)
    for q in range(4):
        x1_ref[q] = h[:, 16 * q:16 * (q + 1)]
    rdeg_ref[...] = deg


def _tc_layer1(xin, mp, ws, wn, b):
    return pl.pallas_call(
        _layer1_body,
        grid=(_GRID,),
        in_specs=[
            pl.BlockSpec((_B, 2), lambda i: (i, 0)),
            pl.BlockSpec((2, _B, 16), lambda i: (0, i, 0)),
            pl.BlockSpec((2, 64), lambda i: (0, 0)),
            pl.BlockSpec((2, 64), lambda i: (0, 0)),
            pl.BlockSpec((1, 64), lambda i: (0, 0)),
        ],
        out_specs=[
            pl.BlockSpec((4, _B, 16), lambda i: (0, i, 0)),
            pl.BlockSpec((_B, 1), lambda i: (i, 0)),
        ],
        out_shape=[
            jax.ShapeDtypeStruct((4, _N, 16), jnp.float32),
            jax.ShapeDtypeStruct((_N, 1), jnp.float32),
        ],
    )(xin, mp, ws, wn, b.reshape(1, 64))


def _layer_body(x_ref, m_ref, r_ref, ws_ref, wn_ref, b_ref, *rest,
                relu, extra):
    if extra:
        ew_ref, o_ref, u16_ref = rest
    else:
        (o_ref,) = rest
    r = r_ref[...]
    xf = jnp.concatenate([x_ref[q] for q in range(4)], axis=1)
    mf = jnp.concatenate([m_ref[q] for q in range(4)], axis=1) / r
    h = _dot(xf, ws_ref[...]) + _dot(mf, wn_ref[...]) + b_ref[...]
    if relu:
        h = jnp.maximum(h, 0.0)
    for q in range(4):
        o_ref[q] = h[:, 16 * q:16 * (q + 1)]
    if extra:
        u16_ref[...] = _dot(h, ew_ref[...])


def _tc_layer(x, m, rdeg, ws, wn, b, relu, extra_w=None):
    extra = extra_w is not None
    in_specs = [
        pl.BlockSpec((4, _B, 16), lambda i: (0, i, 0)),
        pl.BlockSpec((4, _B, 16), lambda i: (0, i, 0)),
        pl.BlockSpec((_B, 1), lambda i: (i, 0)),
        pl.BlockSpec((64, 64), lambda i: (0, 0)),
        pl.BlockSpec((64, 64), lambda i: (0, 0)),
        pl.BlockSpec((1, 64), lambda i: (0, 0)),
    ]
    out_specs = [pl.BlockSpec((4, _B, 16), lambda i: (0, i, 0))]
    out_shape = [jax.ShapeDtypeStruct((4, _N, 16), jnp.float32)]
    args = [x, m, rdeg, ws, wn, b.reshape(1, 64)]
    if extra:
        in_specs.append(pl.BlockSpec((64, 16), lambda i: (0, 0)))
        out_specs.append(pl.BlockSpec((_B, 16), lambda i: (i, 0)))
        out_shape.append(jax.ShapeDtypeStruct((_N, 16), jnp.float32))
        args.append(extra_w)
    res = pl.pallas_call(
        functools.partial(_layer_body, relu=relu, extra=extra),
        grid=(_GRID,),
        in_specs=in_specs,
        out_specs=out_specs,
        out_shape=out_shape,
    )(*args)
    return res if extra else res[0]


def _f1_body(p_ref, mp_ref, r_ref, bat_ref, v_ref, wp_ref, bp_ref,
             logit_ref, gm_ref, gs_ref, vp_ref):
    i = pl.program_id(0)
    m = mp_ref[0] + mp_ref[1]
    agg = m[:, 0:1] / r_ref[...]
    pf = jnp.concatenate([p_ref[q] for q in range(4)], axis=1)
    l = _dot(pf, wp_ref[...]) + agg + bp_ref[0, 0]
    logit_ref[...] = l
    oh = bat_ref[...] == lax.broadcasted_iota(jnp.int32, (1, _G), 1)
    ohf = oh.astype(jnp.float32)

    @pl.when(i == 0)
    def _():
        gm_ref[...] = jnp.full((1, _G), -1e30, jnp.float32)
        gs_ref[...] = jnp.zeros((1, _G), jnp.float32)
        vp_ref[...] = jnp.zeros((_G, 64), jnp.float32)

    bm = jnp.max(jnp.where(oh, l, -1e30), axis=0, keepdims=True)
    gm_old = gm_ref[...]
    gm_new = jnp.maximum(gm_old, bm)
    e = jnp.where(oh, jnp.exp(l - gm_new), 0.0)
    gs_ref[...] = gs_ref[...] * jnp.exp(gm_old - gm_new) + jnp.sum(
        e, axis=0, keepdims=True)
    gm_ref[...] = gm_new
    cdims = (((0,), (0,)), ((), ()))
    vf = jnp.concatenate([v_ref[q] for q in range(4)], axis=1)
    vp_ref[...] = vp_ref[...] + lax.dot_general(
        ohf, vf, cdims, preferred_element_type=jnp.float32,
        precision=lax.Precision.HIGHEST)


def _tc_f1(p1, mp, rdeg, bat, v2, wp, bp):
    return pl.pallas_call(
        _f1_body,
        grid=(_GRID,),
        in_specs=[
            pl.BlockSpec((4, _B, 16), lambda i: (0, i, 0)),
            pl.BlockSpec((2, _B, 16), lambda i: (0, i, 0)),
            pl.BlockSpec((_B, 1), lambda i: (i, 0)),
            pl.BlockSpec((_B, 1), lambda i: (i, 0)),
            pl.BlockSpec((4, _B, 16), lambda i: (0, i, 0)),
            pl.BlockSpec((64, 1), lambda i: (0, 0)),
            pl.BlockSpec((1, 1), lambda i: (0, 0)),
        ],
        out_specs=[
            pl.BlockSpec((_B, 1), lambda i: (i, 0)),
            pl.BlockSpec((1, _G), lambda i: (0, 0)),
            pl.BlockSpec((1, _G), lambda i: (0, 0)),
            pl.BlockSpec((_G, 64), lambda i: (0, 0)),
        ],
        out_shape=[
            jax.ShapeDtypeStruct((_N, 1), jnp.float32),
            jax.ShapeDtypeStruct((1, _G), jnp.float32),
            jax.ShapeDtypeStruct((1, _G), jnp.float32),
            jax.ShapeDtypeStruct((_G, 64), jnp.float32),
        ],
    )(p1, mp, rdeg, bat, v2, wp, bp.reshape(1, 1))


def _f2_body(l_ref, bat_ref, gm_ref, gs_ref, vp_ref, lw_ref, lb_ref,
             pi_ref, val_ref):
    i = pl.program_id(0)
    lse = gm_ref[...] + jnp.log(gs_ref[...])
    oh = bat_ref[...] == lax.broadcasted_iota(jnp.int32, (1, _G), 1)
    pi_ref[...] = l_ref[...] - jnp.sum(
        jnp.where(oh, lse, 0.0), axis=1, keepdims=True)

    @pl.when(i == 0)
    def _():
        z = _dot(vp_ref[...], lw_ref[...]) + lb_ref[0, 0]
        val_ref[...] = 1.0 / (1.0 + jnp.exp(-z))


def _tc_f2(logits, bat, gm, gs, vp, lw, lb):
    return pl.pallas_call(
        _f2_body,
        grid=(_GRID,),
        in_specs=[
            pl.BlockSpec((_B, 1), lambda i: (i, 0)),
            pl.BlockSpec((_B, 1), lambda i: (i, 0)),
            pl.BlockSpec((1, _G), lambda i: (0, 0)),
            pl.BlockSpec((1, _G), lambda i: (0, 0)),
            pl.BlockSpec((_G, 64), lambda i: (0, 0)),
            pl.BlockSpec((64, 1), lambda i: (0, 0)),
            pl.BlockSpec((1, 1), lambda i: (0, 0)),
        ],
        out_specs=[
            pl.BlockSpec((_B, 1), lambda i: (i, 0)),
            pl.BlockSpec((_G, 1), lambda i: (0, 0)),
        ],
        out_shape=[
            jax.ShapeDtypeStruct((_N, 1), jnp.float32),
            jax.ShapeDtypeStruct((_G, 1), jnp.float32),
        ],
    )(logits, bat, gm, gs, vp, lw, lb.reshape(1, 1))


def kernel(x, params, edge_index, batch):
    pred = x[0, 2] == 1.0
    mods = jax.tree.map(lambda a, b: jnp.where(pred, a, b),
                        params["maker"], params["breaker"])
    f32 = jnp.float32
    xin = x[:, :2]
    src = edge_index[0].astype(jnp.int32)
    dst = edge_index[1].astype(jnp.int32)
    bat = batch.astype(jnp.int32).reshape(_N, 1)

    pad = _EPAD - _E
    srcp = jnp.concatenate([src, jnp.zeros((pad,), jnp.int32)])
    dstp = jnp.concatenate([dst, jnp.full((pad,), _N, jnp.int32)])
    src4 = jnp.concatenate(
        [srcp, srcp + _N, srcp + 2 * _N, srcp + 3 * _N]).reshape(4 * _NROW, 128)
    src_b = srcp.reshape(_NROW, 128)
    dst_r = dstp.reshape(_NROW, 128)
    zeros16 = jnp.zeros((_ACC_R, 16), f32)

    body = params["body"]

    g0 = jnp.concatenate(
        [xin, jnp.ones((_N, 1), f32), jnp.zeros((_N, 13), f32)], axis=1)
    m1 = _seg16(zeros16, src_b, dst_r, g0).reshape(2, _N, 16)
    x1, rdeg = _tc_layer1(xin, m1, body[0][0], body[0][1], body[0][2])

    h = x1
    for li in (1, 2, 3):
        m = _seg64(zeros16, src4, dst_r,
                   h.reshape(4 * _N, 16)).reshape(4, _N, 16)
        h = _tc_layer(h, m, rdeg, body[li][0], body[li][1], body[li][2],
                      relu=(li < 3))
    embeds = h

    m5 = _seg64(zeros16, src4, dst_r,
                embeds.reshape(4 * _N, 16)).reshape(4, _N, 16)
    pol, val = mods["policy"], mods["value"]
    extra_w = jnp.pad(pol[1][1], ((0, 0), (0, 15)))
    p1, u16 = _tc_layer(embeds, m5, rdeg, pol[0][0], pol[0][1], pol[0][2],
                        relu=True, extra_w=extra_w)
    v1 = _tc_layer(embeds, m5, rdeg, val[0][0], val[0][1], val[0][2],
                   relu=True)

    m6v = _seg64(zeros16, src4, dst_r,
                 v1.reshape(4 * _N, 16)).reshape(4, _N, 16)
    v2 = _tc_layer(v1, m6v, rdeg, val[1][0], val[1][1], val[1][2], relu=False)

    m6p = _seg16(zeros16, src_b, dst_r, u16).reshape(2, _N, 16)

    logits, gm, gs, vp = _tc_f1(p1, m6p, rdeg, bat, v2,
                                pol[1][0], pol[1][2])
    pi, value = _tc_f2(logits, bat, gm, gs, vp,
                       mods["lin_w"], mods["lin_b"])
    return pi.reshape(-1), value.reshape(-1)


# speedup vs baseline: 1.0017x; 1.0017x over previous
import functools

import jax
import jax.numpy as jnp
from jax import lax
from jax.experimental import pallas as pl
from jax.experimental.pallas import tpu as pltpu
from jax.experimental.pallas import tpu_sc as plsc

_N = 50000
_E = 800000
_G = 64
_EPAD = 819200
_NROW = _EPAD // 128
_ACC_R = 50048
_B = 2000
_GRID = _N // _B

_mesh = plsc.VectorSubcoreMesh(
    core_axis_name="c", subcore_axis_name="s", num_cores=2, num_subcores=16)


def _make_seg(feature_split):
    ch = 10
    if feature_split:
        rows_per_tile = _NROW // 16
        npass = 2
    else:
        rows_per_tile = _NROW // 32
        npass = 1
    nch = rows_per_tile // ch
    zrows = _ACC_R // 16
    orows = 3128
    obase_last = _N - orows

    def body(zeros_h, src_h, dst_h, tab_h, out_h, idx_s, idx_d, rows, acc,
             gsem, ssem):
        c = lax.axis_index("c")
        s = lax.axis_index("s")
        for p in range(npass):
            if feature_split:
                part = c * npass + p
                src_base = part * _NROW + s * rows_per_tile
                dst_base = s * rows_per_tile
            else:
                part = c
                src_base = (c * 16 + s) * rows_per_tile
                dst_base = src_base
            if p > 0:
                plsc.subcore_barrier()
            pltpu.sync_copy(zeros_h.at[pl.ds(s * zrows, zrows)],
                            acc.at[pl.ds(s * zrows, zrows)])
            plsc.subcore_barrier()

            pltpu.sync_copy(src_h.at[pl.ds(src_base, ch)], idx_s.at[0])
            pltpu.sync_copy(dst_h.at[pl.ds(dst_base, ch)], idx_d.at[0])
            for j in range(ch):
                pltpu.async_copy(tab_h.at[idx_s.at[0, j]], rows.at[0, j], gsem)

            @pl.loop(0, nch)
            def _chunk(k, src_base=src_base, dst_base=dst_base):
                pb = lax.rem(k, 2)
                pn = lax.rem(k + 1, 2)
                for j in range(ch):
                    pltpu.make_async_copy(tab_h.at[idx_s.at[pb, j]],
                                          rows.at[pb, j], gsem).wait()
                sdescs = [pltpu.async_copy(rows.at[pb, j],
                                           acc.at[idx_d.at[pb, j]], ssem,
                                           add=True) for j in range(ch)]

                @pl.when(k < nch - 1)
                def _():
                    pltpu.sync_copy(
                        src_h.at[pl.ds(src_base + (k + 1) * ch, ch)],
                        idx_s.at[pn])
                    pltpu.sync_copy(
                        dst_h.at[pl.ds(dst_base + (k + 1) * ch, ch)],
                        idx_d.at[pn])
                    for j in range(ch):
                        pltpu.async_copy(tab_h.at[idx_s.at[pn, j]],
                                         rows.at[pn, j], gsem)

                for d in sdescs:
                    d.wait()

            plsc.subcore_barrier()
            ob = jnp.minimum(s * orows, obase_last)
            pltpu.sync_copy(acc.at[pl.ds(ob, orows)],
                            out_h.at[pl.ds(part * _N + ob, orows)])

    return pl.kernel(
        body,
        out_type=jax.ShapeDtypeStruct(((4 if feature_split else 2) * _N, 16),
                                      jnp.float32),
        mesh=_mesh,
        scratch_types=[
            pltpu.VMEM((2, ch, 128), jnp.int32),
            pltpu.VMEM((2, ch, 128), jnp.int32),
            pltpu.VMEM((2, ch, 128, 16), jnp.float32),
            pltpu.VMEM_SHARED((_ACC_R, 16), jnp.float32),
            pltpu.SemaphoreType.DMA,
            pltpu.SemaphoreType.DMA,
        ],
        compiler_params=pltpu.CompilerParams(use_tc_tiling_on_sc=False),
    )


_seg64 = _make_seg(feature_split=True)
_seg16 = _make_seg(feature_split=False)


def _dot(a, b):
    return jnp.dot(a, b, preferred_element_type=jnp.float32)


def _pre1_body(xin_ref, ws_ref, b_ref, s_ref):
    s_ref[...] = _dot(xin_ref[...], ws_ref[...]) + b_ref[...]


def _tc_pre1(xin, ws, b):
    return pl.pallas_call(
        _pre1_body,
        grid=(_GRID,),
        in_specs=[
            pl.BlockSpec((_B, 2), lambda i: (i, 0)),
            pl.BlockSpec((2, 64), lambda i: (0, 0)),
            pl.BlockSpec((1, 64), lambda i: (0, 0)),
        ],
        out_specs=pl.BlockSpec((_B, 64), lambda i: (i, 0)),
        out_shape=jax.ShapeDtypeStruct((_N, 64), jnp.float32),
    )(xin, ws, b.reshape(1, 64))


def _post1_body(s_ref, mp_ref, wn_ref, x1_ref, deg_ref):
    m = mp_ref[0] + mp_ref[1]
    deg = jnp.maximum(m[:, 2:3], 1.0)
    agg = m[:, 0:2] / deg
    h = jnp.maximum(s_ref[...] + _dot(agg, wn_ref[...]), 0.0)
    for q in range(4):
        x1_ref[q] = h[:, 16 * q:16 * (q + 1)]
    deg_ref[...] = deg


def _tc_post1(s, mp, wn):
    return pl.pallas_call(
        _post1_body,
        grid=(_GRID,),
        in_specs=[
            pl.BlockSpec((_B, 64), lambda i: (i, 0)),
            pl.BlockSpec((2, _B, 16), lambda i: (0, i, 0)),
            pl.BlockSpec((2, 64), lambda i: (0, 0)),
        ],
        out_specs=[
            pl.BlockSpec((4, _B, 16), lambda i: (0, i, 0)),
            pl.BlockSpec((_B, 1), lambda i: (i, 0)),
        ],
        out_shape=[
            jax.ShapeDtypeStruct((4, _N, 16), jnp.float32),
            jax.ShapeDtypeStruct((_N, 1), jnp.float32),
        ],
    )(s, mp, wn)


def _pre_body(x_ref, ws_ref, b_ref, s_ref):
    xf = jnp.concatenate([x_ref[q] for q in range(4)], axis=1)
    s_ref[...] = _dot(xf, ws_ref[...]) + b_ref[...]


def _tc_pre(x, ws, b):
    return pl.pallas_call(
        _pre_body,
        grid=(_GRID,),
        in_specs=[
            pl.BlockSpec((4, _B, 16), lambda i: (0, i, 0)),
            pl.BlockSpec((64, 64), lambda i: (0, 0)),
            pl.BlockSpec((1, 64), lambda i: (0, 0)),
        ],
        out_specs=pl.BlockSpec((_B, 64), lambda i: (i, 0)),
        out_shape=jax.ShapeDtypeStruct((_N, 64), jnp.float32),
    )(x, ws, b.reshape(1, 64))


def _pre2_body(x_ref, wsa_ref, ba_ref, wsb_ref, bb_ref, sa_ref, sb_ref):
    xf = jnp.concatenate([x_ref[q] for q in range(4)], axis=1)
    sa_ref[...] = _dot(xf, wsa_ref[...]) + ba_ref[...]
    sb_ref[...] = _dot(xf, wsb_ref[...]) + bb_ref[...]


def _tc_pre2(x, wsa, ba, wsb, bb):
    return pl.pallas_call(
        _pre2_body,
        grid=(_GRID,),
        in_specs=[
            pl.BlockSpec((4, _B, 16), lambda i: (0, i, 0)),
            pl.BlockSpec((64, 64), lambda i: (0, 0)),
            pl.BlockSpec((1, 64), lambda i: (0, 0)),
            pl.BlockSpec((64, 64), lambda i: (0, 0)),
            pl.BlockSpec((1, 64), lambda i: (0, 0)),
        ],
        out_specs=[
            pl.BlockSpec((_B, 64), lambda i: (i, 0)),
            pl.BlockSpec((_B, 64), lambda i: (i, 0)),
        ],
        out_shape=[
            jax.ShapeDtypeStruct((_N, 64), jnp.float32),
            jax.ShapeDtypeStruct((_N, 64), jnp.float32),
        ],
    )(x, wsa, ba.reshape(1, 64), wsb, bb.reshape(1, 64))


def _post_body(s_ref, m_ref, r_ref, wn_ref, *rest, relu, extra):
    if extra:
        ew_ref, o_ref, u16_ref = rest
    else:
        (o_ref,) = rest
    mf = jnp.concatenate([m_ref[q] for q in range(4)], axis=1) / r_ref[...]
    h = s_ref[...] + _dot(mf, wn_ref[...])
    if relu:
        h = jnp.maximum(h, 0.0)
    for q in range(4):
        o_ref[q] = h[:, 16 * q:16 * (q + 1)]
    if extra:
        u16_ref[...] = _dot(h, ew_ref[...])


def _tc_post(s, m, deg, wn, relu, extra_w=None):
    extra = extra_w is not None
    in_specs = [
        pl.BlockSpec((_B, 64), lambda i: (i, 0)),
        pl.BlockSpec((4, _B, 16), lambda i: (0, i, 0)),
        pl.BlockSpec((_B, 1), lambda i: (i, 0)),
        pl.BlockSpec((64, 64), lambda i: (0, 0)),
    ]
    out_specs = [pl.BlockSpec((4, _B, 16), lambda i: (0, i, 0))]
    out_shape = [jax.ShapeDtypeStruct((4, _N, 16), jnp.float32)]
    args = [s, m, deg, wn]
    if extra:
        in_specs.append(pl.BlockSpec((64, 16), lambda i: (0, 0)))
        out_specs.append(pl.BlockSpec((_B, 16), lambda i: (i, 0)))
        out_shape.append(jax.ShapeDtypeStruct((_N, 16), jnp.float32))
        args.append(extra_w)
    res = pl.pallas_call(
        functools.partial(_post_body, relu=relu, extra=extra),
        grid=(_GRID,),
        in_specs=in_specs,
        out_specs=out_specs,
        out_shape=out_shape,
    )(*args)
    return res if extra else res[0]


def _f1a_body(p_ref, v_ref, wp_ref, bat_ref, lp_ref, vp_ref):
    i = pl.program_id(0)
    pf = jnp.concatenate([p_ref[q] for q in range(4)], axis=1)
    lp_ref[...] = _dot(pf, wp_ref[...])
    oh = bat_ref[...] == lax.broadcasted_iota(jnp.int32, (1, _G), 1)
    ohf = oh.astype(jnp.float32)

    @pl.when(i == 0)
    def _():
        vp_ref[...] = jnp.zeros((_G, 64), jnp.float32)

    cdims = (((0,), (0,)), ((), ()))
    vf = jnp.concatenate([v_ref[q] for q in range(4)], axis=1)
    vp_ref[...] = vp_ref[...] + lax.dot_general(
        ohf, vf, cdims, preferred_element_type=jnp.float32,
        precision=lax.Precision.HIGHEST)


def _tc_f1a(p1, v2, wp, bat):
    return pl.pallas_call(
        _f1a_body,
        grid=(_GRID,),
        in_specs=[
            pl.BlockSpec((4, _B, 16), lambda i: (0, i, 0)),
            pl.BlockSpec((4, _B, 16), lambda i: (0, i, 0)),
            pl.BlockSpec((64, 1), lambda i: (0, 0)),
            pl.BlockSpec((_B, 1), lambda i: (i, 0)),
        ],
        out_specs=[
            pl.BlockSpec((_B, 1), lambda i: (i, 0)),
            pl.BlockSpec((_G, 64), lambda i: (0, 0)),
        ],
        out_shape=[
            jax.ShapeDtypeStruct((_N, 1), jnp.float32),
            jax.ShapeDtypeStruct((_G, 64), jnp.float32),
        ],
    )(p1, v2, wp, bat)


def _f1b_body(lp_ref, mp_ref, r_ref, bat_ref, bp_ref,
              logit_ref, gm_ref, gs_ref):
    i = pl.program_id(0)
    m = mp_ref[0] + mp_ref[1]
    agg = m[:, 0:1] / r_ref[...]
    l = lp_ref[...] + agg + bp_ref[0, 0]
    logit_ref[...] = l
    oh = bat_ref[...] == lax.broadcasted_iota(jnp.int32, (1, _G), 1)

    @pl.when(i == 0)
    def _():
        gm_ref[...] = jnp.full((1, _G), -1e30, jnp.float32)
        gs_ref[...] = jnp.zeros((1, _G), jnp.float32)

    bm = jnp.max(jnp.where(oh, l, -1e30), axis=0, keepdims=True)
    gm_old = gm_ref[...]
    gm_new = jnp.maximum(gm_old, bm)
    e = jnp.where(oh, jnp.exp(l - gm_new), 0.0)
    gs_ref[...] = gs_ref[...] * jnp.exp(gm_old - gm_new) + jnp.sum(
        e, axis=0, keepdims=True)
    gm_ref[...] = gm_new


def _tc_f1b(lp, mp, deg, bat, bp):
    return pl.pallas_call(
        _f1b_body,
        grid=(_GRID,),
        in_specs=[
            pl.BlockSpec((_B, 1), lambda i: (i, 0)),
            pl.BlockSpec((2, _B, 16), lambda i: (0, i, 0)),
            pl.BlockSpec((_B, 1), lambda i: (i, 0)),
            pl.BlockSpec((_B, 1), lambda i: (i, 0)),
            pl.BlockSpec((1, 1), lambda i: (0, 0)),
        ],
        out_specs=[
            pl.BlockSpec((_B, 1), lambda i: (i, 0)),
            pl.BlockSpec((1, _G), lambda i: (0, 0)),
            pl.BlockSpec((1, _G), lambda i: (0, 0)),
        ],
        out_shape=[
            jax.ShapeDtypeStruct((_N, 1), jnp.float32),
            jax.ShapeDtypeStruct((1, _G), jnp.float32),
            jax.ShapeDtypeStruct((1, _G), jnp.float32),
        ],
    )(lp, mp, deg, bat, bp.reshape(1, 1))


def _f2_body(l_ref, bat_ref, gm_ref, gs_ref, vp_ref, lw_ref, lb_ref,
             pi_ref, val_ref):
    i = pl.program_id(0)
    lse = gm_ref[...] + jnp.log(gs_ref[...])
    oh = bat_ref[...] == lax.broadcasted_iota(jnp.int32, (1, _G), 1)
    pi_ref[...] = l_ref[...] - jnp.sum(
        jnp.where(oh, lse, 0.0), axis=1, keepdims=True)

    @pl.when(i == 0)
    def _():
        z = _dot(vp_ref[...], lw_ref[...]) + lb_ref[0, 0]
        val_ref[...] = 1.0 / (1.0 + jnp.exp(-z))


def _tc_f2(logits, bat, gm, gs, vp, lw, lb):
    return pl.pallas_call(
        _f2_body,
        grid=(_GRID,),
        in_specs=[
            pl.BlockSpec((_B, 1), lambda i: (i, 0)),
            pl.BlockSpec((_B, 1), lambda i: (i, 0)),
            pl.BlockSpec((1, _G), lambda i: (0, 0)),
            pl.BlockSpec((1, _G), lambda i: (0, 0)),
            pl.BlockSpec((_G, 64), lambda i: (0, 0)),
            pl.BlockSpec((64, 1), lambda i: (0, 0)),
            pl.BlockSpec((1, 1), lambda i: (0, 0)),
        ],
        out_specs=[
            pl.BlockSpec((_B, 1), lambda i: (i, 0)),
            pl.BlockSpec((_G, 1), lambda i: (0, 0)),
        ],
        out_shape=[
            jax.ShapeDtypeStruct((_N, 1), jnp.float32),
            jax.ShapeDtypeStruct((_G, 1), jnp.float32),
        ],
    )(logits, bat, gm, gs, vp, lw, lb.reshape(1, 1))


def kernel(x, params, edge_index, batch):
    pred = x[0, 2] == 1.0
    mods = jax.tree.map(lambda a, b: jnp.where(pred, a, b),
                        params["maker"], params["breaker"])
    f32 = jnp.float32
    xin = x[:, :2]
    src = edge_index[0].astype(jnp.int32)
    dst = edge_index[1].astype(jnp.int32)
    bat = batch.astype(jnp.int32).reshape(_N, 1)

    pad = _EPAD - _E
    srcp = jnp.concatenate([src, jnp.zeros((pad,), jnp.int32)])
    dstp = jnp.concatenate([dst, jnp.full((pad,), _N, jnp.int32)])
    src4 = jnp.concatenate(
        [srcp, srcp + _N, srcp + 2 * _N, srcp + 3 * _N]).reshape(4 * _NROW, 128)
    src_b = srcp.reshape(_NROW, 128)
    dst_r = dstp.reshape(_NROW, 128)
    zeros16 = jnp.zeros((_ACC_R, 16), f32)

    body = params["body"]

    g0 = jnp.concatenate(
        [xin, jnp.ones((_N, 1), f32), jnp.zeros((_N, 13), f32)], axis=1)
    m1 = _seg16(zeros16, src_b, dst_r, g0).reshape(2, _N, 16)
    s1 = _tc_pre1(xin, body[0][0], body[0][2])
    h, deg = _tc_post1(s1, m1, body[0][1])

    for li in (1, 2, 3):
        m = _seg64(zeros16, src4, dst_r,
                   h.reshape(4 * _N, 16)).reshape(4, _N, 16)
        s = _tc_pre(h, body[li][0], body[li][2])
        h = _tc_post(s, m, deg, body[li][1], relu=(li < 3))
    embeds = h

    m5 = _seg64(zeros16, src4, dst_r,
                embeds.reshape(4 * _N, 16)).reshape(4, _N, 16)
    pol, val = mods["policy"], mods["value"]
    sp, sv = _tc_pre2(embeds, pol[0][0], pol[0][2], val[0][0], val[0][2])
    extra_w = jnp.pad(pol[1][1], ((0, 0), (0, 15)))
    p1, u16 = _tc_post(sp, m5, deg, pol[0][1], relu=True, extra_w=extra_w)
    v1 = _tc_post(sv, m5, deg, val[0][1], relu=True)

    m6p = _seg16(zeros16, src_b, dst_r, u16).reshape(2, _N, 16)
    m6v = _seg64(zeros16, src4, dst_r,
                 v1.reshape(4 * _N, 16)).reshape(4, _N, 16)
    sv2 = _tc_pre(v1, val[1][0], val[1][2])
    v2 = _tc_post(sv2, m6v, deg, val[1][1], relu=False)

    lp, vp = _tc_f1a(p1, v2, pol[1][0], bat)
    logits, gm, gs = _tc_f1b(lp, m6p, deg, bat, pol[1][2])
    pi, value = _tc_f2(logits, bat, gm, gs, vp,
                       mods["lin_w"], mods["lin_b"])
    return pi.reshape(-1), value.reshape(-1)


# speedup vs baseline: 1.0408x; 1.0390x over previous
import functools

import jax
import jax.numpy as jnp
from jax import lax
from jax.experimental import pallas as pl
from jax.experimental.pallas import tpu as pltpu
from jax.experimental.pallas import tpu_sc as plsc

_N = 50000
_E = 800000
_G = 64
_EPAD = 819200
_NROW = _EPAD // 128
_ACC_R = 50048
_B = 2000
_GRID = _N // _B

_mesh = plsc.VectorSubcoreMesh(
    core_axis_name="c", subcore_axis_name="s", num_cores=2, num_subcores=16)


def _make_seg(feature_split):
    ch = 10
    if feature_split:
        rows_per_tile = _NROW // 16
        npass = 2
    else:
        rows_per_tile = _NROW // 32
        npass = 1
    nch = rows_per_tile // ch
    zrows = _ACC_R // 16
    orows = 3128
    obase_last = _N - orows

    def body(zeros_h, src_h, dst_h, tab_h, out_h, idx_s, idx_d, rows, acc,
             gsem, ssem):
        c = lax.axis_index("c")
        s = lax.axis_index("s")
        for p in range(npass):
            if feature_split:
                part = c * npass + p
                src_base = s * rows_per_tile
                dst_base = src_base
                tab_p = tab_h.at[part]
            else:
                part = c
                src_base = (c * 16 + s) * rows_per_tile
                dst_base = src_base
                tab_p = tab_h
            if p > 0:
                plsc.subcore_barrier()
            pltpu.sync_copy(zeros_h.at[pl.ds(s * zrows, zrows)],
                            acc.at[pl.ds(s * zrows, zrows)])
            plsc.subcore_barrier()

            pltpu.sync_copy(src_h.at[pl.ds(src_base, ch)], idx_s.at[0])
            pltpu.sync_copy(dst_h.at[pl.ds(dst_base, ch)], idx_d.at[0])
            for j in range(ch):
                pltpu.async_copy(tab_p.at[idx_s.at[0, j]], rows.at[0, j], gsem)

            @pl.loop(0, nch)
            def _chunk(k, src_base=src_base, dst_base=dst_base, tab_p=tab_p):
                pb = lax.rem(k, 2)
                pn = lax.rem(k + 1, 2)
                for j in range(ch):
                    pltpu.make_async_copy(tab_p.at[idx_s.at[pb, j]],
                                          rows.at[pb, j], gsem).wait()
                sdescs = [pltpu.async_copy(rows.at[pb, j],
                                           acc.at[idx_d.at[pb, j]], ssem,
                                           add=True) for j in range(ch)]

                @pl.when(k < nch - 1)
                def _():
                    pltpu.sync_copy(
                        src_h.at[pl.ds(src_base + (k + 1) * ch, ch)],
                        idx_s.at[pn])
                    pltpu.sync_copy(
                        dst_h.at[pl.ds(dst_base + (k + 1) * ch, ch)],
                        idx_d.at[pn])
                    for j in range(ch):
                        pltpu.async_copy(tab_p.at[idx_s.at[pn, j]],
                                         rows.at[pn, j], gsem)

                for d in sdescs:
                    d.wait()

            plsc.subcore_barrier()
            ob = jnp.minimum(s * orows, obase_last)
            pltpu.sync_copy(acc.at[pl.ds(ob, orows)],
                            out_h.at[part, pl.ds(ob, orows)])

    return pl.kernel(
        body,
        out_type=jax.ShapeDtypeStruct(((4 if feature_split else 2), _N, 16),
                                      jnp.float32),
        mesh=_mesh,
        scratch_types=[
            pltpu.VMEM((2, ch, 128), jnp.int32),
            pltpu.VMEM((2, ch, 128), jnp.int32),
            pltpu.VMEM((2, ch, 128, 16), jnp.float32),
            pltpu.VMEM_SHARED((_ACC_R, 16), jnp.float32),
            pltpu.SemaphoreType.DMA,
            pltpu.SemaphoreType.DMA,
        ],
        compiler_params=pltpu.CompilerParams(use_tc_tiling_on_sc=False),
    )


_seg64 = _make_seg(feature_split=True)
_seg16 = _make_seg(feature_split=False)


def _dot(a, b):
    return jnp.dot(a, b, preferred_element_type=jnp.float32)


def _pre1_body(xin_ref, ws_ref, b_ref, s_ref):
    s_ref[...] = _dot(xin_ref[...], ws_ref[...]) + b_ref[...]


def _tc_pre1(xin, ws, b):
    return pl.pallas_call(
        _pre1_body,
        grid=(_GRID,),
        in_specs=[
            pl.BlockSpec((_B, 2), lambda i: (i, 0)),
            pl.BlockSpec((2, 64), lambda i: (0, 0)),
            pl.BlockSpec((1, 64), lambda i: (0, 0)),
        ],
        out_specs=pl.BlockSpec((_B, 64), lambda i: (i, 0)),
        out_shape=jax.ShapeDtypeStruct((_N, 64), jnp.float32),
    )(xin, ws, b.reshape(1, 64))


def _post1_body(s_ref, mp_ref, wn_ref, x1_ref, deg_ref):
    m = mp_ref[0] + mp_ref[1]
    deg = jnp.maximum(m[:, 2:3], 1.0)
    agg = m[:, 0:2] / deg
    h = jnp.maximum(s_ref[...] + _dot(agg, wn_ref[...]), 0.0)
    for q in range(4):
        x1_ref[q] = h[:, 16 * q:16 * (q + 1)]
    deg_ref[...] = deg


def _tc_post1(s, mp, wn):
    return pl.pallas_call(
        _post1_body,
        grid=(_GRID,),
        in_specs=[
            pl.BlockSpec((_B, 64), lambda i: (i, 0)),
            pl.BlockSpec((2, _B, 16), lambda i: (0, i, 0)),
            pl.BlockSpec((2, 64), lambda i: (0, 0)),
        ],
        out_specs=[
            pl.BlockSpec((4, _B, 16), lambda i: (0, i, 0)),
            pl.BlockSpec((_B, 1), lambda i: (i, 0)),
        ],
        out_shape=[
            jax.ShapeDtypeStruct((4, _N, 16), jnp.float32),
            jax.ShapeDtypeStruct((_N, 1), jnp.float32),
        ],
    )(s, mp, wn)


def _pre_body(x_ref, ws_ref, b_ref, s_ref):
    xf = jnp.concatenate([x_ref[q] for q in range(4)], axis=1)
    s_ref[...] = _dot(xf, ws_ref[...]) + b_ref[...]


def _tc_pre(x, ws, b):
    return pl.pallas_call(
        _pre_body,
        grid=(_GRID,),
        in_specs=[
            pl.BlockSpec((4, _B, 16), lambda i: (0, i, 0)),
            pl.BlockSpec((64, 64), lambda i: (0, 0)),
            pl.BlockSpec((1, 64), lambda i: (0, 0)),
        ],
        out_specs=pl.BlockSpec((_B, 64), lambda i: (i, 0)),
        out_shape=jax.ShapeDtypeStruct((_N, 64), jnp.float32),
    )(x, ws, b.reshape(1, 64))


def _pre2_body(x_ref, wsa_ref, ba_ref, wsb_ref, bb_ref, sa_ref, sb_ref):
    xf = jnp.concatenate([x_ref[q] for q in range(4)], axis=1)
    sa_ref[...] = _dot(xf, wsa_ref[...]) + ba_ref[...]
    sb_ref[...] = _dot(xf, wsb_ref[...]) + bb_ref[...]


def _tc_pre2(x, wsa, ba, wsb, bb):
    return pl.pallas_call(
        _pre2_body,
        grid=(_GRID,),
        in_specs=[
            pl.BlockSpec((4, _B, 16), lambda i: (0, i, 0)),
            pl.BlockSpec((64, 64), lambda i: (0, 0)),
            pl.BlockSpec((1, 64), lambda i: (0, 0)),
            pl.BlockSpec((64, 64), lambda i: (0, 0)),
            pl.BlockSpec((1, 64), lambda i: (0, 0)),
        ],
        out_specs=[
            pl.BlockSpec((_B, 64), lambda i: (i, 0)),
            pl.BlockSpec((_B, 64), lambda i: (i, 0)),
        ],
        out_shape=[
            jax.ShapeDtypeStruct((_N, 64), jnp.float32),
            jax.ShapeDtypeStruct((_N, 64), jnp.float32),
        ],
    )(x, wsa, ba.reshape(1, 64), wsb, bb.reshape(1, 64))


def _post_body(s_ref, m_ref, r_ref, wn_ref, *rest, relu, extra):
    if extra:
        ew_ref, o_ref, u16_ref = rest
    else:
        (o_ref,) = rest
    mf = jnp.concatenate([m_ref[q] for q in range(4)], axis=1) / r_ref[...]
    h = s_ref[...] + _dot(mf, wn_ref[...])
    if relu:
        h = jnp.maximum(h, 0.0)
    for q in range(4):
        o_ref[q] = h[:, 16 * q:16 * (q + 1)]
    if extra:
        u16_ref[...] = _dot(h, ew_ref[...])


def _tc_post(s, m, deg, wn, relu, extra_w=None):
    extra = extra_w is not None
    in_specs = [
        pl.BlockSpec((_B, 64), lambda i: (i, 0)),
        pl.BlockSpec((4, _B, 16), lambda i: (0, i, 0)),
        pl.BlockSpec((_B, 1), lambda i: (i, 0)),
        pl.BlockSpec((64, 64), lambda i: (0, 0)),
    ]
    out_specs = [pl.BlockSpec((4, _B, 16), lambda i: (0, i, 0))]
    out_shape = [jax.ShapeDtypeStruct((4, _N, 16), jnp.float32)]
    args = [s, m, deg, wn]
    if extra:
        in_specs.append(pl.BlockSpec((64, 16), lambda i: (0, 0)))
        out_specs.append(pl.BlockSpec((_B, 16), lambda i: (i, 0)))
        out_shape.append(jax.ShapeDtypeStruct((_N, 16), jnp.float32))
        args.append(extra_w)
    res = pl.pallas_call(
        functools.partial(_post_body, relu=relu, extra=extra),
        grid=(_GRID,),
        in_specs=in_specs,
        out_specs=out_specs,
        out_shape=out_shape,
    )(*args)
    return res if extra else res[0]


def _f1a_body(p_ref, v_ref, wp_ref, bat_ref, lp_ref, vp_ref):
    i = pl.program_id(0)
    pf = jnp.concatenate([p_ref[q] for q in range(4)], axis=1)
    lp_ref[...] = _dot(pf, wp_ref[...])
    oh = bat_ref[...] == lax.broadcasted_iota(jnp.int32, (1, _G), 1)
    ohf = oh.astype(jnp.float32)

    @pl.when(i == 0)
    def _():
        vp_ref[...] = jnp.zeros((_G, 64), jnp.float32)

    cdims = (((0,), (0,)), ((), ()))
    vf = jnp.concatenate([v_ref[q] for q in range(4)], axis=1)
    vp_ref[...] = vp_ref[...] + lax.dot_general(
        ohf, vf, cdims, preferred_element_type=jnp.float32,
        precision=lax.Precision.HIGHEST)


def _tc_f1a(p1, v2, wp, bat):
    return pl.pallas_call(
        _f1a_body,
        grid=(_GRID,),
        in_specs=[
            pl.BlockSpec((4, _B, 16), lambda i: (0, i, 0)),
            pl.BlockSpec((4, _B, 16), lambda i: (0, i, 0)),
            pl.BlockSpec((64, 1), lambda i: (0, 0)),
            pl.BlockSpec((_B, 1), lambda i: (i, 0)),
        ],
        out_specs=[
            pl.BlockSpec((_B, 1), lambda i: (i, 0)),
            pl.BlockSpec((_G, 64), lambda i: (0, 0)),
        ],
        out_shape=[
            jax.ShapeDtypeStruct((_N, 1), jnp.float32),
            jax.ShapeDtypeStruct((_G, 64), jnp.float32),
        ],
    )(p1, v2, wp, bat)


def _f1b_body(lp_ref, mp_ref, r_ref, bat_ref, bp_ref,
              logit_ref, gm_ref, gs_ref):
    i = pl.program_id(0)
    m = mp_ref[0] + mp_ref[1]
    agg = m[:, 0:1] / r_ref[...]
    l = lp_ref[...] + agg + bp_ref[0, 0]
    logit_ref[...] = l
    oh = bat_ref[...] == lax.broadcasted_iota(jnp.int32, (1, _G), 1)

    @pl.when(i == 0)
    def _():
        gm_ref[...] = jnp.full((1, _G), -1e30, jnp.float32)
        gs_ref[...] = jnp.zeros((1, _G), jnp.float32)

    bm = jnp.max(jnp.where(oh, l, -1e30), axis=0, keepdims=True)
    gm_old = gm_ref[...]
    gm_new = jnp.maximum(gm_old, bm)
    e = jnp.where(oh, jnp.exp(l - gm_new), 0.0)
    gs_ref[...] = gs_ref[...] * jnp.exp(gm_old - gm_new) + jnp.sum(
        e, axis=0, keepdims=True)
    gm_ref[...] = gm_new


def _tc_f1b(lp, mp, deg, bat, bp):
    return pl.pallas_call(
        _f1b_body,
        grid=(_GRID,),
        in_specs=[
            pl.BlockSpec((_B, 1), lambda i: (i, 0)),
            pl.BlockSpec((2, _B, 16), lambda i: (0, i, 0)),
            pl.BlockSpec((_B, 1), lambda i: (i, 0)),
            pl.BlockSpec((_B, 1), lambda i: (i, 0)),
            pl.BlockSpec((1, 1), lambda i: (0, 0)),
        ],
        out_specs=[
            pl.BlockSpec((_B, 1), lambda i: (i, 0)),
            pl.BlockSpec((1, _G), lambda i: (0, 0)),
            pl.BlockSpec((1, _G), lambda i: (0, 0)),
        ],
        out_shape=[
            jax.ShapeDtypeStruct((_N, 1), jnp.float32),
            jax.ShapeDtypeStruct((1, _G), jnp.float32),
            jax.ShapeDtypeStruct((1, _G), jnp.float32),
        ],
    )(lp, mp, deg, bat, bp.reshape(1, 1))


def _f2_body(l_ref, bat_ref, gm_ref, gs_ref, vp_ref, lw_ref, lb_ref,
             pi_ref, val_ref):
    i = pl.program_id(0)
    lse = gm_ref[...] + jnp.log(gs_ref[...])
    oh = bat_ref[...] == lax.broadcasted_iota(jnp.int32, (1, _G), 1)
    pi_ref[...] = l_ref[...] - jnp.sum(
        jnp.where(oh, lse, 0.0), axis=1, keepdims=True)

    @pl.when(i == 0)
    def _():
        z = _dot(vp_ref[...], lw_ref[...]) + lb_ref[0, 0]
        val_ref[...] = 1.0 / (1.0 + jnp.exp(-z))


def _tc_f2(logits, bat, gm, gs, vp, lw, lb):
    return pl.pallas_call(
        _f2_body,
        grid=(_GRID,),
        in_specs=[
            pl.BlockSpec((_B, 1), lambda i: (i, 0)),
            pl.BlockSpec((_B, 1), lambda i: (i, 0)),
            pl.BlockSpec((1, _G), lambda i: (0, 0)),
            pl.BlockSpec((1, _G), lambda i: (0, 0)),
            pl.BlockSpec((_G, 64), lambda i: (0, 0)),
            pl.BlockSpec((64, 1), lambda i: (0, 0)),
            pl.BlockSpec((1, 1), lambda i: (0, 0)),
        ],
        out_specs=[
            pl.BlockSpec((_B, 1), lambda i: (i, 0)),
            pl.BlockSpec((_G, 1), lambda i: (0, 0)),
        ],
        out_shape=[
            jax.ShapeDtypeStruct((_N, 1), jnp.float32),
            jax.ShapeDtypeStruct((_G, 1), jnp.float32),
        ],
    )(logits, bat, gm, gs, vp, lw, lb.reshape(1, 1))


def kernel(x, params, edge_index, batch):
    pred = x[0, 2] == 1.0
    mods = jax.tree.map(lambda a, b: jnp.where(pred, a, b),
                        params["maker"], params["breaker"])
    f32 = jnp.float32
    xin = x[:, :2]
    src = edge_index[0].astype(jnp.int32)
    dst = edge_index[1].astype(jnp.int32)
    bat = batch.astype(jnp.int32).reshape(_N, 1)

    pad = _EPAD - _E
    srcp = jnp.concatenate([src, jnp.zeros((pad,), jnp.int32)])
    dstp = jnp.concatenate([dst, jnp.full((pad,), _N, jnp.int32)])
    src_b = srcp.reshape(_NROW, 128)
    dst_r = dstp.reshape(_NROW, 128)
    zeros16 = jnp.zeros((_ACC_R, 16), f32)

    body = params["body"]

    g0 = jnp.concatenate(
        [xin, jnp.ones((_N, 1), f32), jnp.zeros((_N, 13), f32)], axis=1)
    m1 = _seg16(zeros16, src_b, dst_r, g0)
    s1 = _tc_pre1(xin, body[0][0], body[0][2])
    h, deg = _tc_post1(s1, m1, body[0][1])

    for li in (1, 2, 3):
        m = _seg64(zeros16, src_b, dst_r, h)
        s = _tc_pre(h, body[li][0], body[li][2])
        h = _tc_post(s, m, deg, body[li][1], relu=(li < 3))
    embeds = h

    m5 = _seg64(zeros16, src_b, dst_r, embeds)
    pol, val = mods["policy"], mods["value"]
    sp, sv = _tc_pre2(embeds, pol[0][0], pol[0][2], val[0][0], val[0][2])
    extra_w = jnp.pad(pol[1][1], ((0, 0), (0, 15)))
    p1, u16 = _tc_post(sp, m5, deg, pol[0][1], relu=True, extra_w=extra_w)
    v1 = _tc_post(sv, m5, deg, val[0][1], relu=True)

    m6p = _seg16(zeros16, src_b, dst_r, u16)
    m6v = _seg64(zeros16, src_b, dst_r, v1)
    sv2 = _tc_pre(v1, val[1][0], val[1][2])
    v2 = _tc_post(sv2, m6v, deg, val[1][1], relu=False)

    lp, vp = _tc_f1a(p1, v2, pol[1][0], bat)
    logits, gm, gs = _tc_f1b(lp, m6p, deg, bat, pol[1][2])
    pi, value = _tc_f2(logits, bat, gm, gs, vp,
                       mods["lin_w"], mods["lin_b"])
    return pi.reshape(-1), value.reshape(-1)
